# Initial kernel scaffold; baseline (speedup 1.0000x reference)
import functools

import jax
import jax.numpy as jnp
from jax import lax
from jax.experimental import pallas as pl
from jax.experimental.pallas import tpu as pltpu
from jax.experimental.pallas import tpu_sc as plsc

N = 10000
E = 320000
D_IN = 128
H = 64
NUM_OUT = 64
L = 10

NC = 2
NS = 16
N_PAD = 10240
RPT = N_PAD // NS

BATCH = 128
BT = 80
E_PAD = NC * NS * BT * BATCH
CB = 8
NCHUNK = BT // CB

_mesh = plsc.VectorSubcoreMesh(core_axis_name="c", subcore_axis_name="s")


@functools.partial(
    pl.kernel,
    mesh=_mesh,
    out_type=jax.ShapeDtypeStruct((NC, N_PAD, 16), jnp.float32),
    scratch_types=[
        pltpu.VMEM((BT, BATCH), jnp.int32),
        pltpu.VMEM((BATCH, 16), jnp.float32),
        pltpu.VMEM_SHARED((N_PAD, 16), jnp.float32),
    ],
)
def _deg_kernel(dst_hbm, ones_hbm, z_hbm, out_hbm, dst_v, ones_v, deg_sh):
    c = lax.axis_index("c")
    s = lax.axis_index("s")
    wid = s * NC + c
    pltpu.sync_copy(z_hbm.at[pl.ds(s * RPT, RPT)], deg_sh.at[pl.ds(s * RPT, RPT)])
    pltpu.sync_copy(ones_hbm, ones_v)
    pltpu.sync_copy(dst_hbm.at[pl.ds(wid * BT, BT)], dst_v)
    plsc.subcore_barrier()

    def body(j, carry):
        pltpu.sync_copy(ones_v, deg_sh.at[dst_v.at[j]], add=True)
        return carry

    lax.fori_loop(0, BT, body, 0)
    plsc.subcore_barrier()
    pltpu.sync_copy(deg_sh.at[pl.ds(s * RPT, RPT)],
                    out_hbm.at[c, pl.ds(s * RPT, RPT)])


@functools.partial(
    pl.kernel,
    mesh=_mesh,
    out_type=jax.ShapeDtypeStruct((NC, N_PAD, H), jnp.float32),
    scratch_types=[
        pltpu.VMEM((BT, BATCH), jnp.int32),
        pltpu.VMEM((BT, BATCH), jnp.int32),
        pltpu.VMEM((CB * BATCH, H), jnp.float32),
        pltpu.VMEM_SHARED((N_PAD, H), jnp.float32),
        pltpu.SemaphoreType.DMA,
    ],
)
def _mp_kernel(t_hbm, src_hbm, dst_hbm, z_hbm, out_hbm,
               src_v, dst_v, rows_v, agg_sh, sem):
    c = lax.axis_index("c")
    s = lax.axis_index("s")
    wid = s * NC + c
    pltpu.sync_copy(z_hbm.at[pl.ds(s * RPT, RPT)], agg_sh.at[pl.ds(s * RPT, RPT)])
    pltpu.sync_copy(src_hbm.at[pl.ds(wid * BT, BT)], src_v)
    pltpu.sync_copy(dst_hbm.at[pl.ds(wid * BT, BT)], dst_v)
    plsc.subcore_barrier()

    def chunk(i, carry):
        descs = []
        for j in range(CB):
            d = pltpu.async_copy(t_hbm.at[src_v.at[i * CB + j]],
                                 rows_v.at[pl.ds(j * BATCH, BATCH)], sem)
            descs.append(d)
        for d in descs:
            d.wait()
        for j in range(CB):
            pltpu.sync_copy(rows_v.at[pl.ds(j * BATCH, BATCH)],
                            agg_sh.at[dst_v.at[i * CB + j]], add=True)
        return carry

    lax.fori_loop(0, NCHUNK, chunk, 0)
    plsc.subcore_barrier()
    pltpu.sync_copy(agg_sh.at[pl.ds(s * RPT, RPT)],
                    out_hbm.at[c, pl.ds(s * RPT, RPT)])


_BN = 2048
_GRID = N_PAD // _BN


def _dinv_block(d0, d1):
    deg = d0[:, 0:1] + d1[:, 0:1] + 1.0
    return lax.rsqrt(deg)


def _first_body(x_ref, w_ref, d0_ref, d1_ref, o_ref):
    dinv = _dinv_block(d0_ref[...], d1_ref[...])
    h = jnp.dot(x_ref[...], w_ref[...], preferred_element_type=jnp.float32)
    o_ref[...] = h * dinv


def _mid_body(p0_ref, p1_ref, tp_ref, d0_ref, d1_ref, b_ref, w_ref, o_ref):
    dinv = _dinv_block(d0_ref[...], d1_ref[...])
    agg = (p0_ref[...] + p1_ref[...] + tp_ref[...]) * dinv + b_ref[...]
    h = jnp.maximum(agg, 0.0)
    o_ref[...] = jnp.dot(h, w_ref[...], preferred_element_type=jnp.float32) * dinv


def _last_body(p0_ref, p1_ref, tp_ref, d0_ref, d1_ref, b_ref, w_ref, bl_ref,
               o_ref):
    dinv = _dinv_block(d0_ref[...], d1_ref[...])
    agg = (p0_ref[...] + p1_ref[...] + tp_ref[...]) * dinv + b_ref[...]
    h = jnp.maximum(agg, 0.0)
    o_ref[...] = (jnp.dot(h, w_ref[...], preferred_element_type=jnp.float32)
                  + bl_ref[...])


def _row_spec(width):
    return pl.BlockSpec((_BN, width), lambda i: (i, 0))


def _full_spec(r, cc):
    return pl.BlockSpec((r, cc), lambda i: (0, 0))


def _tc_first(x, w0, d0, d1):
    return pl.pallas_call(
        _first_body,
        grid=(_GRID,),
        in_specs=[_row_spec(D_IN), _full_spec(D_IN, H), _row_spec(16),
                  _row_spec(16)],
        out_specs=_row_spec(H),
        out_shape=jax.ShapeDtypeStruct((N_PAD, H), jnp.float32),
    )(x, w0, d0, d1)


def _tc_mid(p0, p1, tp, d0, d1, b, w):
    return pl.pallas_call(
        _mid_body,
        grid=(_GRID,),
        in_specs=[_row_spec(H), _row_spec(H), _row_spec(H), _row_spec(16),
                  _row_spec(16), _full_spec(1, H), _full_spec(H, H)],
        out_specs=_row_spec(H),
        out_shape=jax.ShapeDtypeStruct((N_PAD, H), jnp.float32),
    )(p0, p1, tp, d0, d1, b, w)


def _tc_last(p0, p1, tp, d0, d1, b, wl, bl):
    return pl.pallas_call(
        _last_body,
        grid=(_GRID,),
        in_specs=[_row_spec(H), _row_spec(H), _row_spec(H), _row_spec(16),
                  _row_spec(16), _full_spec(1, H), _full_spec(H, NUM_OUT),
                  _full_spec(1, NUM_OUT)],
        out_specs=_row_spec(NUM_OUT),
        out_shape=jax.ShapeDtypeStruct((N_PAD, NUM_OUT), jnp.float32),
    )(p0, p1, tp, d0, d1, b, wl, bl)


def kernel(x, edge_index, W0, b0, W_rest, b_rest, W_logits, b_logits):
    src = edge_index[0].astype(jnp.int32)
    dst = edge_index[1].astype(jnp.int32)
    npad = E_PAD - E
    pad_iota = lax.iota(jnp.int32, npad)
    src_p = jnp.concatenate([src, pad_iota % N]).reshape(E_PAD // BATCH, BATCH)
    dst_p = jnp.concatenate([dst, N + pad_iota % (N_PAD - N)]).reshape(
        E_PAD // BATCH, BATCH)

    z16 = jnp.zeros((N_PAD, 16), jnp.float32)
    z64 = jnp.zeros((N_PAD, H), jnp.float32)
    ones16 = jnp.ones((BATCH, 16), jnp.float32)

    deg_parts = _deg_kernel(dst_p, ones16, z16)
    d0 = deg_parts[0]
    d1 = deg_parts[1]

    xp = jnp.pad(x, ((0, N_PAD - N), (0, 0)))
    t = _tc_first(xp, W0, d0, d1)

    biases = [b0] + [b_rest[i] for i in range(L - 1)]
    weights = [W_rest[i] for i in range(L - 1)]

    for i in range(L - 1):
        parts = _mp_kernel(t, src_p, dst_p, z64)
        b2 = biases[i].reshape(1, H)
        t = _tc_mid(parts[0], parts[1], t, d0, d1, b2, weights[i])

    parts = _mp_kernel(t, src_p, dst_p, z64)
    blast = biases[L - 1].reshape(1, H)
    logits = _tc_last(parts[0], parts[1], t, d0, d1, blast,
                      W_logits, b_logits.reshape(1, NUM_OUT))
    return logits[:N]


# speedup vs baseline: 4.7590x; 4.7590x over previous
import functools

import jax
import jax.numpy as jnp
from jax import lax
from jax.experimental import pallas as pl
from jax.experimental.pallas import tpu as pltpu
from jax.experimental.pallas import tpu_sc as plsc

N = 10000
E = 320000
D_IN = 128
H = 64
HP = 128
NUM_OUT = 64
L = 10

NC = 2
NS = 16
N_PAD = 10240
RPT = N_PAD // NS

BATCH = 128
BT = 160
E_PAD = NS * BT * BATCH
CB = 4
NCHUNK = BT // CB

_mesh = plsc.VectorSubcoreMesh(core_axis_name="c", subcore_axis_name="s", num_cores=NC)


QUARTER = N_PAD // 4
NDUM = 256
AGG_R = QUARTER + NDUM
RPT_A = AGG_R // NS


def _make_mp(k):
    @functools.partial(
        pl.kernel,
        mesh=_mesh,
        compiler_params=pltpu.CompilerParams(use_tc_tiling_on_sc=False),
        out_type=(jax.ShapeDtypeStruct((QUARTER, H), jnp.float32),
                  jax.ShapeDtypeStruct((QUARTER, H), jnp.float32)),
        scratch_types=[
            pltpu.VMEM((BT, BATCH), jnp.int32),
            pltpu.VMEM((BT, BATCH), jnp.int32),
            pltpu.VMEM((CB * BATCH, HP), jnp.float32),
            pltpu.VMEM_SHARED((AGG_R, HP), jnp.float32),
            pltpu.SemaphoreType.DMA,
        ],
    )
    def _mp_kernel(t_hbm, src_hbm, dst_hbm, z_hbm, out0_hbm, out1_hbm,
                   src_v, dst_v, rows_v, agg_sh, sem):
        c = lax.axis_index("c")
        s = lax.axis_index("s")
        pltpu.sync_copy(z_hbm.at[pl.ds(s * RPT_A, RPT_A)],
                        agg_sh.at[pl.ds(s * RPT_A, RPT_A)])
        pltpu.sync_copy(src_hbm.at[pl.ds(s * BT, BT)], src_v)
        pltpu.sync_copy(dst_hbm.at[pl.ds(s * BT, BT)], dst_v)

        base = (2 * k + c) * QUARTER
        iota16 = lax.iota(jnp.int32, 16)

        def remap(b, carry):
            for j in range(BATCH // 16):
                v = dst_v[b, pl.ds(j * 16, 16)]
                local = v - base
                oob = (local < 0) | (local >= QUARTER)
                dummy = QUARTER + ((iota16 + (b * 8 + j) * 16) & (NDUM - 1))
                dst_v[b, pl.ds(j * 16, 16)] = jnp.where(oob, dummy, local)
            return carry

        lax.fori_loop(0, BT, remap, 0)
        plsc.subcore_barrier()

        def chunk(i, carry):
            descs = []
            for j in range(CB):
                d = pltpu.async_copy(t_hbm.at[src_v.at[i * CB + j]],
                                     rows_v.at[pl.ds(j * BATCH, BATCH)], sem)
                descs.append(d)
            for d in descs:
                d.wait()
            for j in range(CB):
                pltpu.sync_copy(rows_v.at[pl.ds(j * BATCH, BATCH)],
                                agg_sh.at[dst_v.at[i * CB + j]], add=True)
            return carry

        lax.fori_loop(0, NCHUNK, chunk, 0)
        plsc.subcore_barrier()

        rps = QUARTER // NS
        @pl.when(c == 0)
        def _():
            pltpu.sync_copy(agg_sh.at[pl.ds(s * rps, rps), pl.ds(0, H)],
                            out0_hbm.at[pl.ds(s * rps, rps)])

        @pl.when(c == 1)
        def _():
            pltpu.sync_copy(agg_sh.at[pl.ds(s * rps, rps), pl.ds(0, H)],
                            out1_hbm.at[pl.ds(s * rps, rps)])

    return _mp_kernel


_mp0 = _make_mp(0)
_mp1 = _make_mp(1)


def _mp_full(t, src_p, dst_p, z):
    q0, q1 = _mp0(t, src_p, dst_p, z)
    q2, q3 = _mp1(t, src_p, dst_p, z)
    return jnp.concatenate([q0, q1, q2, q3], axis=0)


_BN = 2048
_GRID = N_PAD // _BN


def _dinv_block(d0):
    deg = d0[:, 0:1] + 1.0
    return lax.rsqrt(deg)


_ZBLK = (_BN, H)


def _first_body(x_ref, w_ref, d0_ref, o_ref):
    dinv = _dinv_block(d0_ref[...])
    h = jnp.dot(x_ref[...], w_ref[...], preferred_element_type=jnp.float32)
    o_ref[...] = h * dinv


def _mid_body(p0_ref, tp_ref, d0_ref, b_ref, w_ref, o_ref):
    dinv = _dinv_block(d0_ref[...])
    agg = (p0_ref[...] + tp_ref[:, :H]) * dinv + b_ref[...]
    h = jnp.maximum(agg, 0.0)
    hw = jnp.dot(h, w_ref[...], preferred_element_type=jnp.float32) * dinv
    o_ref[...] = jnp.concatenate([hw, jnp.zeros(_ZBLK, jnp.float32)], axis=1)


def _last_body(p0_ref, tp_ref, d0_ref, b_ref, w_ref, bl_ref,
               o_ref):
    dinv = _dinv_block(d0_ref[...])
    agg = (p0_ref[...] + tp_ref[:, :H]) * dinv + b_ref[...]
    h = jnp.maximum(agg, 0.0)
    o_ref[...] = (jnp.dot(h, w_ref[...], preferred_element_type=jnp.float32)
                  + bl_ref[...])


def _row_spec(width):
    return pl.BlockSpec((_BN, width), lambda i: (i, 0))


def _full_spec(r, cc):
    return pl.BlockSpec((r, cc), lambda i: (0, 0))


def _tc_first(x, w0, d0):
    return pl.pallas_call(
        _first_body,
        grid=(_GRID,),
        in_specs=[_row_spec(D_IN), _full_spec(D_IN, HP), _row_spec(H)],
        out_specs=_row_spec(HP),
        out_shape=jax.ShapeDtypeStruct((N_PAD, HP), jnp.float32),
    )(x, w0, d0)


def _tc_mid(p0, tp, d0, b, w):
    return pl.pallas_call(
        _mid_body,
        grid=(_GRID,),
        in_specs=[_row_spec(H), _row_spec(HP), _row_spec(H),
                  _full_spec(1, H), _full_spec(H, H)],
        out_specs=_row_spec(HP),
        out_shape=jax.ShapeDtypeStruct((N_PAD, HP), jnp.float32),
    )(p0, tp, d0, b, w)


def _tc_last(p0, tp, d0, b, wl, bl):
    return pl.pallas_call(
        _last_body,
        grid=(_GRID,),
        in_specs=[_row_spec(H), _row_spec(HP), _row_spec(H),
                  _full_spec(1, H), _full_spec(H, NUM_OUT), _full_spec(1, NUM_OUT)],
        out_specs=_row_spec(NUM_OUT),
        out_shape=jax.ShapeDtypeStruct((N_PAD, NUM_OUT), jnp.float32),
    )(p0, tp, d0, b, wl, bl)


def _pad2(a, r, cc):
    return jnp.pad(a, ((0, r - a.shape[0]), (0, cc - a.shape[1])))


def kernel(x, edge_index, W0, b0, W_rest, b_rest, W_logits, b_logits):
    src = edge_index[0].astype(jnp.int32)
    dst = edge_index[1].astype(jnp.int32)
    npad = E_PAD - E
    pad_iota = lax.iota(jnp.int32, npad)
    src_p = jnp.concatenate([src, pad_iota % N]).reshape(E_PAD // BATCH, BATCH)
    dst_p = jnp.concatenate([dst, N + pad_iota % (N_PAD - N)]).reshape(
        E_PAD // BATCH, BATCH)

    z = jnp.zeros((AGG_R, HP), jnp.float32)
    ones_t = jnp.ones((N_PAD, HP), jnp.float32)

    d0 = _mp_full(ones_t, src_p, dst_p, z)

    xp = jnp.pad(x, ((0, N_PAD - N), (0, 0)))
    t = _tc_first(xp, _pad2(W0, D_IN, HP), d0)

    biases = [b0] + [b_rest[i] for i in range(L - 1)]
    weights = [W_rest[i] for i in range(L - 1)]

    for i in range(L - 1):
        p = _mp_full(t, src_p, dst_p, z)
        t = _tc_mid(p, t, d0, biases[i].reshape(1, H), weights[i])

    p = _mp_full(t, src_p, dst_p, z)
    logits = _tc_last(p, t, d0, biases[L - 1].reshape(1, H),
                      W_logits, b_logits.reshape(1, NUM_OUT))
    return logits[:N]


# speedup vs baseline: 9.2551x; 1.9448x over previous
import functools

import jax
import jax.numpy as jnp
from jax import lax
from jax.experimental import pallas as pl
from jax.experimental.pallas import tpu as pltpu
from jax.experimental.pallas import tpu_sc as plsc

N = 10000
E = 320000
D_IN = 128
H = 64
HP = 128
NUM_OUT = 64
L = 10

NC = 2
NS = 16
N_PAD = 10240
RPT = N_PAD // NS

BATCH = 128
BT = 160
E_PAD = NS * BT * BATCH
CB = 2
CBB = CB * BATCH

_mesh = plsc.VectorSubcoreMesh(core_axis_name="c", subcore_axis_name="s", num_cores=NC)


QUARTER = N_PAD // 4
NDUM = 256
AGG_R = QUARTER + NDUM
RPT_A = AGG_R // NS


def _make_mp(k):
    @functools.partial(
        pl.kernel,
        mesh=_mesh,
        compiler_params=pltpu.CompilerParams(use_tc_tiling_on_sc=False, needs_layout_passes=False),
        out_type=(jax.ShapeDtypeStruct((QUARTER, H), jnp.float32),
                  jax.ShapeDtypeStruct((QUARTER, H), jnp.float32)),
        scratch_types=[
            pltpu.VMEM((BT + 3, BATCH), jnp.int32),
            pltpu.VMEM((BT + 3, BATCH), jnp.int32),
            pltpu.VMEM((CB * BATCH, HP), jnp.float32),
            pltpu.VMEM_SHARED((AGG_R, HP), jnp.float32),
            pltpu.SemaphoreType.DMA,
        ],
    )
    def _mp_kernel(t_hbm, src_hbm, dst_hbm, z_hbm, out0_hbm, out1_hbm,
                   src_v, dst_v, rows_v, agg_sh, sem):
        c = lax.axis_index("c")
        s = lax.axis_index("s")
        pltpu.sync_copy(z_hbm.at[pl.ds(s * RPT_A, RPT_A)],
                        agg_sh.at[pl.ds(s * RPT_A, RPT_A)])
        pltpu.sync_copy(src_hbm.at[pl.ds(s * BT, BT)], src_v.at[pl.ds(0, BT)])
        pltpu.sync_copy(dst_hbm.at[pl.ds(s * BT, BT)], dst_v.at[pl.ds(0, BT)])

        base = (2 * k + c) * QUARTER
        iota16 = lax.iota(jnp.int32, 16)

        def compact(b, cnt):
            for j in range(BATCH // 16):
                v_dst = dst_v[b, pl.ds(j * 16, 16)]
                v_src = src_v[b, pl.ds(j * 16, 16)]
                local = v_dst - base
                m = (local >= 0) & (local < QUARTER)
                mi = jnp.where(m, jnp.int32(1), jnp.int32(0))
                pos = cnt + plsc.cumsum(mi) - 1
                r = lax.shift_right_logical(pos, 7)
                q = pos & (BATCH - 1)
                plsc.store_scatter(dst_v, [r, q], local, mask=m)
                plsc.store_scatter(src_v, [r, q], v_src, mask=m)
                cnt = cnt + jnp.sum(mi)
            return cnt

        cnt = lax.fori_loop(0, BT, compact, jnp.int32(0))

        for j in range(CBB // 16):
            dummy = QUARTER + ((iota16 + j * 16) & (NDUM - 1))
            tpos = cnt + j * 16 + iota16
            tr = lax.shift_right_logical(tpos, 7)
            tq = tpos & (BATCH - 1)
            plsc.store_scatter(dst_v, [tr, tq], dummy)
            plsc.store_scatter(src_v, [tr, tq], (iota16 + j * 16) * 8)

        nb = (cnt + CBB - 1) // CBB
        plsc.subcore_barrier()

        def chunk(i, carry):
            descs = []
            for j in range(CB):
                d = pltpu.async_copy(t_hbm.at[src_v.at[i * CB + j]],
                                     rows_v.at[pl.ds(j * BATCH, BATCH)], sem)
                descs.append(d)
            for d in descs:
                d.wait()
            for j in range(CB):
                pltpu.sync_copy(rows_v.at[pl.ds(j * BATCH, BATCH)],
                                agg_sh.at[dst_v.at[i * CB + j]], add=True)
            return carry

        lax.fori_loop(0, nb, chunk, 0)
        plsc.subcore_barrier()

        rps = QUARTER // NS
        @pl.when(c == 0)
        def _():
            pltpu.sync_copy(agg_sh.at[pl.ds(s * rps, rps), pl.ds(0, H)],
                            out0_hbm.at[pl.ds(s * rps, rps)])

        @pl.when(c == 1)
        def _():
            pltpu.sync_copy(agg_sh.at[pl.ds(s * rps, rps), pl.ds(0, H)],
                            out1_hbm.at[pl.ds(s * rps, rps)])

    return _mp_kernel


_mp0 = _make_mp(0)
_mp1 = _make_mp(1)


def _mp_full(t, src_p, dst_p, z):
    q0, q1 = _mp0(t, src_p, dst_p, z)
    q2, q3 = _mp1(t, src_p, dst_p, z)
    return jnp.concatenate([q0, q1, q2, q3], axis=0)


_BN = 2048
_GRID = N_PAD // _BN


def _dinv_block(d0):
    deg = d0[:, 0:1] + 1.0
    return lax.rsqrt(deg)


_ZBLK = (_BN, H)


def _first_body(x_ref, w_ref, d0_ref, o_ref):
    dinv = _dinv_block(d0_ref[...])
    h = jnp.dot(x_ref[...], w_ref[...], preferred_element_type=jnp.float32)
    o_ref[...] = h * dinv


def _mid_body(p0_ref, tp_ref, d0_ref, b_ref, w_ref, o_ref):
    dinv = _dinv_block(d0_ref[...])
    agg = (p0_ref[...] + tp_ref[:, :H]) * dinv + b_ref[...]
    h = jnp.maximum(agg, 0.0)
    hw = jnp.dot(h, w_ref[...], preferred_element_type=jnp.float32) * dinv
    o_ref[...] = jnp.concatenate([hw, jnp.zeros(_ZBLK, jnp.float32)], axis=1)


def _last_body(p0_ref, tp_ref, d0_ref, b_ref, w_ref, bl_ref,
               o_ref):
    dinv = _dinv_block(d0_ref[...])
    agg = (p0_ref[...] + tp_ref[:, :H]) * dinv + b_ref[...]
    h = jnp.maximum(agg, 0.0)
    o_ref[...] = (jnp.dot(h, w_ref[...], preferred_element_type=jnp.float32)
                  + bl_ref[...])


def _row_spec(width):
    return pl.BlockSpec((_BN, width), lambda i: (i, 0))


def _full_spec(r, cc):
    return pl.BlockSpec((r, cc), lambda i: (0, 0))


def _tc_first(x, w0, d0):
    return pl.pallas_call(
        _first_body,
        grid=(_GRID,),
        in_specs=[_row_spec(D_IN), _full_spec(D_IN, HP), _row_spec(H)],
        out_specs=_row_spec(HP),
        out_shape=jax.ShapeDtypeStruct((N_PAD, HP), jnp.float32),
    )(x, w0, d0)


def _tc_mid(p0, tp, d0, b, w):
    return pl.pallas_call(
        _mid_body,
        grid=(_GRID,),
        in_specs=[_row_spec(H), _row_spec(HP), _row_spec(H),
                  _full_spec(1, H), _full_spec(H, H)],
        out_specs=_row_spec(HP),
        out_shape=jax.ShapeDtypeStruct((N_PAD, HP), jnp.float32),
    )(p0, tp, d0, b, w)


def _tc_last(p0, tp, d0, b, wl, bl):
    return pl.pallas_call(
        _last_body,
        grid=(_GRID,),
        in_specs=[_row_spec(H), _row_spec(HP), _row_spec(H),
                  _full_spec(1, H), _full_spec(H, NUM_OUT), _full_spec(1, NUM_OUT)],
        out_specs=_row_spec(NUM_OUT),
        out_shape=jax.ShapeDtypeStruct((N_PAD, NUM_OUT), jnp.float32),
    )(p0, tp, d0, b, wl, bl)


def _pad2(a, r, cc):
    return jnp.pad(a, ((0, r - a.shape[0]), (0, cc - a.shape[1])))


def kernel(x, edge_index, W0, b0, W_rest, b_rest, W_logits, b_logits):
    src = edge_index[0].astype(jnp.int32)
    dst = edge_index[1].astype(jnp.int32)
    npad = E_PAD - E
    pad_iota = lax.iota(jnp.int32, npad)
    src_p = jnp.concatenate([src, pad_iota % N]).reshape(E_PAD // BATCH, BATCH)
    dst_p = jnp.concatenate([dst, N + pad_iota % (N_PAD - N)]).reshape(
        E_PAD // BATCH, BATCH)

    z = jnp.zeros((AGG_R, HP), jnp.float32)
    ones_t = jnp.ones((N_PAD, HP), jnp.float32)

    d0 = _mp_full(ones_t, src_p, dst_p, z)

    xp = jnp.pad(x, ((0, N_PAD - N), (0, 0)))
    t = _tc_first(xp, _pad2(W0, D_IN, HP), d0)

    biases = [b0] + [b_rest[i] for i in range(L - 1)]
    weights = [W_rest[i] for i in range(L - 1)]

    for i in range(L - 1):
        p = _mp_full(t, src_p, dst_p, z)
        t = _tc_mid(p, t, d0, biases[i].reshape(1, H), weights[i])

    p = _mp_full(t, src_p, dst_p, z)
    logits = _tc_last(p, t, d0, biases[L - 1].reshape(1, H),
                      W_logits, b_logits.reshape(1, NUM_OUT))
    return logits[:N]


# speedup vs baseline: 11.5697x; 1.2501x over previous
import functools

import jax
import jax.numpy as jnp
from jax import lax
from jax.experimental import pallas as pl
from jax.experimental.pallas import tpu as pltpu
from jax.experimental.pallas import tpu_sc as plsc

N = 10000
E = 320000
D_IN = 128
H = 64
HP = 128
NUM_OUT = 64
L = 10

NC = 2
NS = 16
N_PAD = 10240
RPT = N_PAD // NS

BATCH = 128
BT = 160
E_PAD = NS * BT * BATCH
CB = 2
CBB = CB * BATCH

_mesh = plsc.VectorSubcoreMesh(core_axis_name="c", subcore_axis_name="s", num_cores=NC)


HALFR = N_PAD // 2
NDUM = 256
AGG_R = HALFR + NDUM
RPT_A = AGG_R // NS


def _make_mp():
    @functools.partial(
        pl.kernel,
        mesh=_mesh,
        compiler_params=pltpu.CompilerParams(use_tc_tiling_on_sc=False, needs_layout_passes=False),
        out_type=(jax.ShapeDtypeStruct((HALFR, H), jnp.float32),
                  jax.ShapeDtypeStruct((HALFR, H), jnp.float32)),
        scratch_types=[
            pltpu.VMEM((BT + 3, BATCH), jnp.int32),
            pltpu.VMEM((BT + 3, BATCH), jnp.int32),
            pltpu.VMEM((CB * BATCH, HP), jnp.float32),
            pltpu.VMEM_SHARED((AGG_R, HP), jnp.float32),
            pltpu.SemaphoreType.DMA,
        ],
    )
    def _mp_kernel(t_hbm, src_hbm, dst_hbm, z_hbm, out0_hbm, out1_hbm,
                   src_v, dst_v, rows_v, agg_sh, sem):
        c = lax.axis_index("c")
        s = lax.axis_index("s")
        pltpu.sync_copy(z_hbm.at[pl.ds(s * RPT_A, RPT_A)],
                        agg_sh.at[pl.ds(s * RPT_A, RPT_A)])
        pltpu.sync_copy(src_hbm.at[pl.ds(s * BT, BT)], src_v.at[pl.ds(0, BT)])
        pltpu.sync_copy(dst_hbm.at[pl.ds(s * BT, BT)], dst_v.at[pl.ds(0, BT)])

        base = c * HALFR
        iota16 = lax.iota(jnp.int32, 16)

        def compact(b, cnt):
            for j in range(BATCH // 16):
                v_dst = dst_v[b, pl.ds(j * 16, 16)]
                v_src = src_v[b, pl.ds(j * 16, 16)]
                local = v_dst - base
                m = (local >= 0) & (local < HALFR)
                mi = jnp.where(m, jnp.int32(1), jnp.int32(0))
                pos = cnt + plsc.cumsum(mi) - 1
                r = lax.shift_right_logical(pos, 7)
                q = pos & (BATCH - 1)
                plsc.store_scatter(dst_v, [r, q], local, mask=m)
                plsc.store_scatter(src_v, [r, q], v_src, mask=m)
                cnt = cnt + jnp.sum(mi)
            return cnt

        cnt = lax.fori_loop(0, BT, compact, jnp.int32(0))

        for j in range(CBB // 16):
            dummy = HALFR + ((iota16 + j * 16) & (NDUM - 1))
            tpos = cnt + j * 16 + iota16
            tr = lax.shift_right_logical(tpos, 7)
            tq = tpos & (BATCH - 1)
            plsc.store_scatter(dst_v, [tr, tq], dummy)
            plsc.store_scatter(src_v, [tr, tq], (iota16 + j * 16) * 8)

        nb = (cnt + CBB - 1) // CBB
        plsc.subcore_barrier()

        def chunk(i, carry):
            descs = []
            for j in range(CB):
                d = pltpu.async_copy(t_hbm.at[src_v.at[i * CB + j]],
                                     rows_v.at[pl.ds(j * BATCH, BATCH)], sem)
                descs.append(d)
            for d in descs:
                d.wait()
            for j in range(CB):
                pltpu.sync_copy(rows_v.at[pl.ds(j * BATCH, BATCH)],
                                agg_sh.at[dst_v.at[i * CB + j]], add=True)
            return carry

        lax.fori_loop(0, nb, chunk, 0)
        plsc.subcore_barrier()

        rps = HALFR // NS
        @pl.when(c == 0)
        def _():
            pltpu.sync_copy(agg_sh.at[pl.ds(s * rps, rps), pl.ds(0, H)],
                            out0_hbm.at[pl.ds(s * rps, rps)])

        @pl.when(c == 1)
        def _():
            pltpu.sync_copy(agg_sh.at[pl.ds(s * rps, rps), pl.ds(0, H)],
                            out1_hbm.at[pl.ds(s * rps, rps)])

    return _mp_kernel


_mp = _make_mp()


def _mp_full(t, src_p, dst_p, z):
    h0, h1 = _mp(t, src_p, dst_p, z)
    return jnp.concatenate([h0, h1], axis=0)


_BN = 2048
_GRID = N_PAD // _BN


def _dinv_block(d0):
    deg = d0[:, 0:1] + 1.0
    return lax.rsqrt(deg)


_ZBLK = (_BN, H)


def _first_body(x_ref, w_ref, d0_ref, o_ref):
    dinv = _dinv_block(d0_ref[...])
    h = jnp.dot(x_ref[...], w_ref[...], preferred_element_type=jnp.float32)
    o_ref[...] = h * dinv


def _mid_body(p0_ref, tp_ref, d0_ref, b_ref, w_ref, o_ref):
    dinv = _dinv_block(d0_ref[...])
    agg = (p0_ref[...] + tp_ref[:, :H]) * dinv + b_ref[...]
    h = jnp.maximum(agg, 0.0)
    hw = jnp.dot(h, w_ref[...], preferred_element_type=jnp.float32) * dinv
    o_ref[...] = jnp.concatenate([hw, jnp.zeros(_ZBLK, jnp.float32)], axis=1)


def _last_body(p0_ref, tp_ref, d0_ref, b_ref, w_ref, bl_ref,
               o_ref):
    dinv = _dinv_block(d0_ref[...])
    agg = (p0_ref[...] + tp_ref[:, :H]) * dinv + b_ref[...]
    h = jnp.maximum(agg, 0.0)
    o_ref[...] = (jnp.dot(h, w_ref[...], preferred_element_type=jnp.float32)
                  + bl_ref[...])


def _row_spec(width):
    return pl.BlockSpec((_BN, width), lambda i: (i, 0))


def _full_spec(r, cc):
    return pl.BlockSpec((r, cc), lambda i: (0, 0))


def _tc_first(x, w0, d0):
    return pl.pallas_call(
        _first_body,
        grid=(_GRID,),
        in_specs=[_row_spec(D_IN), _full_spec(D_IN, HP), _row_spec(H)],
        out_specs=_row_spec(HP),
        out_shape=jax.ShapeDtypeStruct((N_PAD, HP), jnp.float32),
    )(x, w0, d0)


def _tc_mid(p0, tp, d0, b, w):
    return pl.pallas_call(
        _mid_body,
        grid=(_GRID,),
        in_specs=[_row_spec(H), _row_spec(HP), _row_spec(H),
                  _full_spec(1, H), _full_spec(H, H)],
        out_specs=_row_spec(HP),
        out_shape=jax.ShapeDtypeStruct((N_PAD, HP), jnp.float32),
    )(p0, tp, d0, b, w)


def _tc_last(p0, tp, d0, b, wl, bl):
    return pl.pallas_call(
        _last_body,
        grid=(_GRID,),
        in_specs=[_row_spec(H), _row_spec(HP), _row_spec(H),
                  _full_spec(1, H), _full_spec(H, NUM_OUT), _full_spec(1, NUM_OUT)],
        out_specs=_row_spec(NUM_OUT),
        out_shape=jax.ShapeDtypeStruct((N_PAD, NUM_OUT), jnp.float32),
    )(p0, tp, d0, b, wl, bl)


def _pad2(a, r, cc):
    return jnp.pad(a, ((0, r - a.shape[0]), (0, cc - a.shape[1])))


def kernel(x, edge_index, W0, b0, W_rest, b_rest, W_logits, b_logits):
    src = edge_index[0].astype(jnp.int32)
    dst = edge_index[1].astype(jnp.int32)
    npad = E_PAD - E
    pad_iota = lax.iota(jnp.int32, npad)
    src_p = jnp.concatenate([src, pad_iota % N]).reshape(E_PAD // BATCH, BATCH)
    dst_p = jnp.concatenate([dst, N + pad_iota % (N_PAD - N)]).reshape(
        E_PAD // BATCH, BATCH)

    z = jnp.zeros((AGG_R, HP), jnp.float32)
    ones_t = jnp.ones((N_PAD, HP), jnp.float32)

    d0 = _mp_full(ones_t, src_p, dst_p, z)

    xp = jnp.pad(x, ((0, N_PAD - N), (0, 0)))
    t = _tc_first(xp, _pad2(W0, D_IN, HP), d0)

    biases = [b0] + [b_rest[i] for i in range(L - 1)]
    weights = [W_rest[i] for i in range(L - 1)]

    for i in range(L - 1):
        p = _mp_full(t, src_p, dst_p, z)
        t = _tc_mid(p, t, d0, biases[i].reshape(1, H), weights[i])

    p = _mp_full(t, src_p, dst_p, z)
    logits = _tc_last(p, t, d0, biases[L - 1].reshape(1, H),
                      W_logits, b_logits.reshape(1, NUM_OUT))
    return logits[:N]


# speedup vs baseline: 16.5868x; 1.4336x over previous
import functools

import jax
import jax.numpy as jnp
from jax import lax
from jax.experimental import pallas as pl
from jax.experimental.pallas import tpu as pltpu
from jax.experimental.pallas import tpu_sc as plsc

N = 10000
E = 320000
D_IN = 128
H = 64
HP = 128
NUM_OUT = 64
L = 10

NC = 2
NS = 16
N_PAD = 10240
RPT = N_PAD // NS

BATCH = 128
BT = 160
E_PAD = NS * BT * BATCH
CB = 2
CBB = CB * BATCH

_mesh = plsc.VectorSubcoreMesh(core_axis_name="c", subcore_axis_name="s", num_cores=NC)


HALFR = N_PAD // 2
NDUM = 256
AGG_R = HALFR + NDUM
RPT_A = AGG_R // NS


def _make_mp():
    @functools.partial(
        pl.kernel,
        mesh=_mesh,
        compiler_params=pltpu.CompilerParams(use_tc_tiling_on_sc=False, needs_layout_passes=False),
        out_type=(jax.ShapeDtypeStruct((HALFR, H), jnp.float32),
                  jax.ShapeDtypeStruct((HALFR, H), jnp.float32)),
        scratch_types=[
            pltpu.VMEM((BT + 3, BATCH), jnp.int32),
            pltpu.VMEM((BT + 3, BATCH), jnp.int32),
            pltpu.VMEM((CB * BATCH, H), jnp.float32),
            pltpu.VMEM_SHARED((AGG_R, H), jnp.float32),
            pltpu.SemaphoreType.DMA,
        ],
    )
    def _mp_kernel(t_hbm, src_hbm, dst_hbm, z_hbm, out0_hbm, out1_hbm,
                   src_v, dst_v, rows_v, agg_sh, sem):
        c = lax.axis_index("c")
        s = lax.axis_index("s")
        pltpu.sync_copy(z_hbm.at[pl.ds(s * RPT_A, RPT_A)],
                        agg_sh.at[pl.ds(s * RPT_A, RPT_A)])
        pltpu.sync_copy(src_hbm.at[pl.ds(s * BT, BT)], src_v.at[pl.ds(0, BT)])
        pltpu.sync_copy(dst_hbm.at[pl.ds(s * BT, BT)], dst_v.at[pl.ds(0, BT)])

        base = c * HALFR
        iota16 = lax.iota(jnp.int32, 16)

        def compact(b, cnt):
            for j in range(BATCH // 16):
                v_dst = dst_v[b, pl.ds(j * 16, 16)]
                v_src = src_v[b, pl.ds(j * 16, 16)]
                local = v_dst - base
                m = (local >= 0) & (local < HALFR)
                mi = jnp.where(m, jnp.int32(1), jnp.int32(0))
                pos = cnt + plsc.cumsum(mi) - 1
                r = lax.shift_right_logical(pos, 7)
                q = pos & (BATCH - 1)
                plsc.store_scatter(dst_v, [r, q], local, mask=m)
                plsc.store_scatter(src_v, [r, q], v_src, mask=m)
                cnt = cnt + jnp.sum(mi)
            return cnt

        cnt = lax.fori_loop(0, BT, compact, jnp.int32(0))

        for j in range(CBB // 16):
            dummy = HALFR + ((iota16 + j * 16) & (NDUM - 1))
            tpos = cnt + j * 16 + iota16
            tr = lax.shift_right_logical(tpos, 7)
            tq = tpos & (BATCH - 1)
            plsc.store_scatter(dst_v, [tr, tq], dummy)
            plsc.store_scatter(src_v, [tr, tq], (iota16 + j * 16) * 8)

        nb = (cnt + CBB - 1) // CBB
        plsc.subcore_barrier()

        def chunk(i, carry):
            descs = []
            for j in range(CB):
                d = pltpu.async_copy(t_hbm.at[src_v.at[i * CB + j]],
                                     rows_v.at[pl.ds(j * BATCH, BATCH)], sem)
                descs.append(d)
            for d in descs:
                d.wait()
            for j in range(CB):
                pltpu.sync_copy(rows_v.at[pl.ds(j * BATCH, BATCH)],
                                agg_sh.at[dst_v.at[i * CB + j]], add=True)
            return carry

        lax.fori_loop(0, nb, chunk, 0)
        plsc.subcore_barrier()

        rps = HALFR // NS
        @pl.when(c == 0)
        def _():
            pltpu.sync_copy(agg_sh.at[pl.ds(s * rps, rps)],
                            out0_hbm.at[pl.ds(s * rps, rps)])

        @pl.when(c == 1)
        def _():
            pltpu.sync_copy(agg_sh.at[pl.ds(s * rps, rps)],
                            out1_hbm.at[pl.ds(s * rps, rps)])

    return _mp_kernel


_mp = _make_mp()


def _mp_full(t, src_p, dst_p, z):
    h0, h1 = _mp(t, src_p, dst_p, z)
    return jnp.concatenate([h0, h1], axis=0)


_BN = 2048
_GRID = N_PAD // _BN


def _dinv_block(d0):
    deg = d0[:, 0:1] + 1.0
    return lax.rsqrt(deg)


_ZBLK = (_BN, H)


def _first_body(x_ref, w_ref, d0_ref, o_ref):
    dinv = _dinv_block(d0_ref[...])
    h = jnp.dot(x_ref[...], w_ref[...], preferred_element_type=jnp.float32)
    o_ref[...] = h * dinv


def _mid_body(p0_ref, tp_ref, d0_ref, b_ref, w_ref, o_ref):
    dinv = _dinv_block(d0_ref[...])
    agg = (p0_ref[...] + tp_ref[...]) * dinv + b_ref[...]
    h = jnp.maximum(agg, 0.0)
    o_ref[...] = jnp.dot(h, w_ref[...], preferred_element_type=jnp.float32) * dinv


def _last_body(p0_ref, tp_ref, d0_ref, b_ref, w_ref, bl_ref,
               o_ref):
    dinv = _dinv_block(d0_ref[...])
    agg = (p0_ref[...] + tp_ref[...]) * dinv + b_ref[...]
    h = jnp.maximum(agg, 0.0)
    o_ref[...] = (jnp.dot(h, w_ref[...], preferred_element_type=jnp.float32)
                  + bl_ref[...])


def _row_spec(width):
    return pl.BlockSpec((_BN, width), lambda i: (i, 0))


def _full_spec(r, cc):
    return pl.BlockSpec((r, cc), lambda i: (0, 0))


def _tc_first(x, w0, d0):
    return pl.pallas_call(
        _first_body,
        grid=(_GRID,),
        in_specs=[_row_spec(D_IN), _full_spec(D_IN, H), _row_spec(H)],
        out_specs=_row_spec(H),
        out_shape=jax.ShapeDtypeStruct((N_PAD, H), jnp.float32),
    )(x, w0, d0)


def _tc_mid(p0, tp, d0, b, w):
    return pl.pallas_call(
        _mid_body,
        grid=(_GRID,),
        in_specs=[_row_spec(H), _row_spec(H), _row_spec(H),
                  _full_spec(1, H), _full_spec(H, H)],
        out_specs=_row_spec(H),
        out_shape=jax.ShapeDtypeStruct((N_PAD, H), jnp.float32),
    )(p0, tp, d0, b, w)


def _tc_last(p0, tp, d0, b, wl, bl):
    return pl.pallas_call(
        _last_body,
        grid=(_GRID,),
        in_specs=[_row_spec(H), _row_spec(H), _row_spec(H),
                  _full_spec(1, H), _full_spec(H, NUM_OUT), _full_spec(1, NUM_OUT)],
        out_specs=_row_spec(NUM_OUT),
        out_shape=jax.ShapeDtypeStruct((N_PAD, NUM_OUT), jnp.float32),
    )(p0, tp, d0, b, wl, bl)


def _pad2(a, r, cc):
    return jnp.pad(a, ((0, r - a.shape[0]), (0, cc - a.shape[1])))


def kernel(x, edge_index, W0, b0, W_rest, b_rest, W_logits, b_logits):
    src = edge_index[0].astype(jnp.int32)
    dst = edge_index[1].astype(jnp.int32)
    npad = E_PAD - E
    pad_iota = lax.iota(jnp.int32, npad)
    src_p = jnp.concatenate([src, pad_iota % N]).reshape(E_PAD // BATCH, BATCH)
    dst_p = jnp.concatenate([dst, N + pad_iota % (N_PAD - N)]).reshape(
        E_PAD // BATCH, BATCH)

    z = jnp.zeros((AGG_R, H), jnp.float32)
    ones_t = jnp.ones((N_PAD, H), jnp.float32)

    d0 = _mp_full(ones_t, src_p, dst_p, z)

    xp = jnp.pad(x, ((0, N_PAD - N), (0, 0)))
    t = _tc_first(xp, W0, d0)

    biases = [b0] + [b_rest[i] for i in range(L - 1)]
    weights = [W_rest[i] for i in range(L - 1)]

    for i in range(L - 1):
        p = _mp_full(t, src_p, dst_p, z)
        t = _tc_mid(p, t, d0, biases[i].reshape(1, H), weights[i])

    p = _mp_full(t, src_p, dst_p, z)
    logits = _tc_last(p, t, d0, biases[L - 1].reshape(1, H),
                      W_logits, b_logits.reshape(1, NUM_OUT))
    return logits[:N]


# speedup vs baseline: 17.8063x; 1.0735x over previous
import functools

import jax
import jax.numpy as jnp
from jax import lax
from jax.experimental import pallas as pl
from jax.experimental.pallas import tpu as pltpu
from jax.experimental.pallas import tpu_sc as plsc

N = 10000
E = 320000
D_IN = 128
H = 64
HP = 128
NUM_OUT = 64
L = 10

NC = 2
NS = 16
N_PAD = 10240
RPT = N_PAD // NS

BATCH = 128
BT = 160
E_PAD = NS * BT * BATCH
CB = 4
CBB = CB * BATCH

_mesh = plsc.VectorSubcoreMesh(core_axis_name="c", subcore_axis_name="s", num_cores=NC)


HALFR = N_PAD // 2
NDUM = 256
AGG_R = HALFR + NDUM
RPT_A = AGG_R // NS


def _make_mp():
    @functools.partial(
        pl.kernel,
        mesh=_mesh,
        compiler_params=pltpu.CompilerParams(use_tc_tiling_on_sc=False, needs_layout_passes=False),
        out_type=(jax.ShapeDtypeStruct((HALFR, H), jnp.float32),
                  jax.ShapeDtypeStruct((HALFR, H), jnp.float32)),
        scratch_types=[
            pltpu.VMEM((BT + 3, BATCH), jnp.int32),
            pltpu.VMEM((BT + 3, BATCH), jnp.int32),
            pltpu.VMEM((CB * BATCH, H), jnp.float32),
            pltpu.VMEM_SHARED((AGG_R, H), jnp.float32),
            pltpu.SemaphoreType.DMA,
        ],
    )
    def _mp_kernel(t_hbm, src_hbm, dst_hbm, z_hbm, out0_hbm, out1_hbm,
                   src_v, dst_v, rows_v, agg_sh, sem):
        c = lax.axis_index("c")
        s = lax.axis_index("s")
        pltpu.sync_copy(z_hbm.at[pl.ds(s * RPT_A, RPT_A)],
                        agg_sh.at[pl.ds(s * RPT_A, RPT_A)])
        pltpu.sync_copy(src_hbm.at[pl.ds(s * BT, BT)], src_v.at[pl.ds(0, BT)])
        pltpu.sync_copy(dst_hbm.at[pl.ds(s * BT, BT)], dst_v.at[pl.ds(0, BT)])

        base = c * HALFR
        iota16 = lax.iota(jnp.int32, 16)

        def compact(b, cnt):
            for j in range(BATCH // 16):
                v_dst = dst_v[b, pl.ds(j * 16, 16)]
                v_src = src_v[b, pl.ds(j * 16, 16)]
                local = v_dst - base
                m = (local >= 0) & (local < HALFR)
                mi = jnp.where(m, jnp.int32(1), jnp.int32(0))
                pos = cnt + plsc.cumsum(mi) - 1
                r = lax.shift_right_logical(pos, 7)
                q = pos & (BATCH - 1)
                plsc.store_scatter(dst_v, [r, q], local, mask=m)
                plsc.store_scatter(src_v, [r, q], v_src, mask=m)
                cnt = cnt + jnp.sum(mi)
            return cnt

        cnt = lax.fori_loop(0, BT, compact, jnp.int32(0))

        for j in range(CBB // 16):
            dummy = HALFR + ((iota16 + j * 16) & (NDUM - 1))
            tpos = cnt + j * 16 + iota16
            tr = lax.shift_right_logical(tpos, 7)
            tq = tpos & (BATCH - 1)
            plsc.store_scatter(dst_v, [tr, tq], dummy)
            plsc.store_scatter(src_v, [tr, tq], (iota16 + j * 16) * 8)

        nb = (cnt + CBB - 1) // CBB
        plsc.subcore_barrier()

        def chunk(i, carry):
            descs = []
            for j in range(CB):
                d = pltpu.async_copy(t_hbm.at[src_v.at[i * CB + j]],
                                     rows_v.at[pl.ds(j * BATCH, BATCH)], sem)
                descs.append(d)
            for d in descs:
                d.wait()
            for j in range(CB):
                pltpu.sync_copy(rows_v.at[pl.ds(j * BATCH, BATCH)],
                                agg_sh.at[dst_v.at[i * CB + j]], add=True)
            return carry

        lax.fori_loop(0, nb, chunk, 0)
        plsc.subcore_barrier()

        rps = HALFR // NS
        @pl.when(c == 0)
        def _():
            pltpu.sync_copy(agg_sh.at[pl.ds(s * rps, rps)],
                            out0_hbm.at[pl.ds(s * rps, rps)])

        @pl.when(c == 1)
        def _():
            pltpu.sync_copy(agg_sh.at[pl.ds(s * rps, rps)],
                            out1_hbm.at[pl.ds(s * rps, rps)])

    return _mp_kernel


_mp = _make_mp()


def _mp_full(t, src_p, dst_p, z):
    h0, h1 = _mp(t, src_p, dst_p, z)
    return jnp.concatenate([h0, h1], axis=0)


_BN = 2048
_GRID = N_PAD // _BN


def _dinv_block(d0):
    deg = d0[:, 0:1] + 1.0
    return lax.rsqrt(deg)


_ZBLK = (_BN, H)


def _first_body(x_ref, w_ref, d0_ref, o_ref):
    dinv = _dinv_block(d0_ref[...])
    h = jnp.dot(x_ref[...], w_ref[...], preferred_element_type=jnp.float32)
    o_ref[...] = h * dinv


def _mid_body(p0_ref, tp_ref, d0_ref, b_ref, w_ref, o_ref):
    dinv = _dinv_block(d0_ref[...])
    agg = (p0_ref[...] + tp_ref[...]) * dinv + b_ref[...]
    h = jnp.maximum(agg, 0.0)
    o_ref[...] = jnp.dot(h, w_ref[...], preferred_element_type=jnp.float32) * dinv


def _last_body(p0_ref, tp_ref, d0_ref, b_ref, w_ref, bl_ref,
               o_ref):
    dinv = _dinv_block(d0_ref[...])
    agg = (p0_ref[...] + tp_ref[...]) * dinv + b_ref[...]
    h = jnp.maximum(agg, 0.0)
    o_ref[...] = (jnp.dot(h, w_ref[...], preferred_element_type=jnp.float32)
                  + bl_ref[...])


def _row_spec(width):
    return pl.BlockSpec((_BN, width), lambda i: (i, 0))


def _full_spec(r, cc):
    return pl.BlockSpec((r, cc), lambda i: (0, 0))


def _tc_first(x, w0, d0):
    return pl.pallas_call(
        _first_body,
        grid=(_GRID,),
        in_specs=[_row_spec(D_IN), _full_spec(D_IN, H), _row_spec(H)],
        out_specs=_row_spec(H),
        out_shape=jax.ShapeDtypeStruct((N_PAD, H), jnp.float32),
    )(x, w0, d0)


def _tc_mid(p0, tp, d0, b, w):
    return pl.pallas_call(
        _mid_body,
        grid=(_GRID,),
        in_specs=[_row_spec(H), _row_spec(H), _row_spec(H),
                  _full_spec(1, H), _full_spec(H, H)],
        out_specs=_row_spec(H),
        out_shape=jax.ShapeDtypeStruct((N_PAD, H), jnp.float32),
    )(p0, tp, d0, b, w)


def _tc_last(p0, tp, d0, b, wl, bl):
    return pl.pallas_call(
        _last_body,
        grid=(_GRID,),
        in_specs=[_row_spec(H), _row_spec(H), _row_spec(H),
                  _full_spec(1, H), _full_spec(H, NUM_OUT), _full_spec(1, NUM_OUT)],
        out_specs=_row_spec(NUM_OUT),
        out_shape=jax.ShapeDtypeStruct((N_PAD, NUM_OUT), jnp.float32),
    )(p0, tp, d0, b, wl, bl)


def _pad2(a, r, cc):
    return jnp.pad(a, ((0, r - a.shape[0]), (0, cc - a.shape[1])))


def kernel(x, edge_index, W0, b0, W_rest, b_rest, W_logits, b_logits):
    src = edge_index[0].astype(jnp.int32)
    dst = edge_index[1].astype(jnp.int32)
    npad = E_PAD - E
    pad_iota = lax.iota(jnp.int32, npad)
    src_p = jnp.concatenate([src, pad_iota % N]).reshape(E_PAD // BATCH, BATCH)
    dst_p = jnp.concatenate([dst, N + pad_iota % (N_PAD - N)]).reshape(
        E_PAD // BATCH, BATCH)

    z = jnp.zeros((AGG_R, H), jnp.float32)
    ones_t = jnp.ones((N_PAD, H), jnp.float32)

    d0 = _mp_full(ones_t, src_p, dst_p, z)

    xp = jnp.pad(x, ((0, N_PAD - N), (0, 0)))
    t = _tc_first(xp, W0, d0)

    biases = [b0] + [b_rest[i] for i in range(L - 1)]
    weights = [W_rest[i] for i in range(L - 1)]

    for i in range(L - 1):
        p = _mp_full(t, src_p, dst_p, z)
        t = _tc_mid(p, t, d0, biases[i].reshape(1, H), weights[i])

    p = _mp_full(t, src_p, dst_p, z)
    logits = _tc_last(p, t, d0, biases[L - 1].reshape(1, H),
                      W_logits, b_logits.reshape(1, NUM_OUT))
    return logits[:N]


# speedup vs baseline: 18.6046x; 1.0448x over previous
import functools

import jax
import jax.numpy as jnp
from jax import lax
from jax.experimental import pallas as pl
from jax.experimental.pallas import tpu as pltpu
from jax.experimental.pallas import tpu_sc as plsc

N = 10000
E = 320000
D_IN = 128
H = 64
HP = 128
NUM_OUT = 64
L = 10

NC = 2
NS = 16
N_PAD = 10240
RPT = N_PAD // NS

BATCH = 128
BT = 160
E_PAD = NS * BT * BATCH
CB = 8
CBB = CB * BATCH

_mesh = plsc.VectorSubcoreMesh(core_axis_name="c", subcore_axis_name="s", num_cores=NC)


HALFR = N_PAD // 2
NDUM = 256
AGG_R = HALFR + NDUM
RPT_A = AGG_R // NS


def _make_mp():
    @functools.partial(
        pl.kernel,
        mesh=_mesh,
        compiler_params=pltpu.CompilerParams(use_tc_tiling_on_sc=False, needs_layout_passes=False),
        out_type=(jax.ShapeDtypeStruct((HALFR, H), jnp.float32),
                  jax.ShapeDtypeStruct((HALFR, H), jnp.float32)),
        scratch_types=[
            pltpu.VMEM((BT + 3, BATCH), jnp.int32),
            pltpu.VMEM((BT + 3, BATCH), jnp.int32),
            pltpu.VMEM((CB * BATCH, H), jnp.float32),
            pltpu.VMEM_SHARED((AGG_R, H), jnp.float32),
            pltpu.SemaphoreType.DMA,
        ],
    )
    def _mp_kernel(t_hbm, src_hbm, dst_hbm, z_hbm, out0_hbm, out1_hbm,
                   src_v, dst_v, rows_v, agg_sh, sem):
        c = lax.axis_index("c")
        s = lax.axis_index("s")
        pltpu.sync_copy(z_hbm.at[pl.ds(s * RPT_A, RPT_A)],
                        agg_sh.at[pl.ds(s * RPT_A, RPT_A)])
        pltpu.sync_copy(src_hbm.at[pl.ds(s * BT, BT)], src_v.at[pl.ds(0, BT)])
        pltpu.sync_copy(dst_hbm.at[pl.ds(s * BT, BT)], dst_v.at[pl.ds(0, BT)])

        base = c * HALFR
        iota16 = lax.iota(jnp.int32, 16)

        def compact(b, cnt):
            for j in range(BATCH // 16):
                v_dst = dst_v[b, pl.ds(j * 16, 16)]
                v_src = src_v[b, pl.ds(j * 16, 16)]
                local = v_dst - base
                m = (local >= 0) & (local < HALFR)
                mi = jnp.where(m, jnp.int32(1), jnp.int32(0))
                pos = cnt + plsc.cumsum(mi) - 1
                r = lax.shift_right_logical(pos, 7)
                q = pos & (BATCH - 1)
                plsc.store_scatter(dst_v, [r, q], local, mask=m)
                plsc.store_scatter(src_v, [r, q], v_src, mask=m)
                cnt = cnt + jnp.sum(mi)
            return cnt

        cnt = lax.fori_loop(0, BT, compact, jnp.int32(0))

        for j in range(CBB // 16):
            dummy = HALFR + ((iota16 + j * 16) & (NDUM - 1))
            tpos = cnt + j * 16 + iota16
            tr = lax.shift_right_logical(tpos, 7)
            tq = tpos & (BATCH - 1)
            plsc.store_scatter(dst_v, [tr, tq], dummy)
            plsc.store_scatter(src_v, [tr, tq], (iota16 + j * 16) * 8)

        nb = (cnt + CBB - 1) // CBB
        plsc.subcore_barrier()

        def chunk(i, carry):
            descs = []
            for j in range(CB):
                d = pltpu.async_copy(t_hbm.at[src_v.at[i * CB + j]],
                                     rows_v.at[pl.ds(j * BATCH, BATCH)], sem)
                descs.append(d)
            for d in descs:
                d.wait()
            for j in range(CB):
                pltpu.sync_copy(rows_v.at[pl.ds(j * BATCH, BATCH)],
                                agg_sh.at[dst_v.at[i * CB + j]], add=True)
            return carry

        lax.fori_loop(0, nb, chunk, 0)
        plsc.subcore_barrier()

        rps = HALFR // NS
        @pl.when(c == 0)
        def _():
            pltpu.sync_copy(agg_sh.at[pl.ds(s * rps, rps)],
                            out0_hbm.at[pl.ds(s * rps, rps)])

        @pl.when(c == 1)
        def _():
            pltpu.sync_copy(agg_sh.at[pl.ds(s * rps, rps)],
                            out1_hbm.at[pl.ds(s * rps, rps)])

    return _mp_kernel


_mp = _make_mp()


def _mp_full(t, src_p, dst_p, z):
    h0, h1 = _mp(t, src_p, dst_p, z)
    return jnp.concatenate([h0, h1], axis=0)


_BN = 2048
_GRID = N_PAD // _BN


def _dinv_block(d0):
    deg = d0[:, 0:1] + 1.0
    return lax.rsqrt(deg)


_ZBLK = (_BN, H)


def _first_body(x_ref, w_ref, d0_ref, o_ref):
    dinv = _dinv_block(d0_ref[...])
    h = jnp.dot(x_ref[...], w_ref[...], preferred_element_type=jnp.float32)
    o_ref[...] = h * dinv


def _mid_body(p0_ref, tp_ref, d0_ref, b_ref, w_ref, o_ref):
    dinv = _dinv_block(d0_ref[...])
    agg = (p0_ref[...] + tp_ref[...]) * dinv + b_ref[...]
    h = jnp.maximum(agg, 0.0)
    o_ref[...] = jnp.dot(h, w_ref[...], preferred_element_type=jnp.float32) * dinv


def _last_body(p0_ref, tp_ref, d0_ref, b_ref, w_ref, bl_ref,
               o_ref):
    dinv = _dinv_block(d0_ref[...])
    agg = (p0_ref[...] + tp_ref[...]) * dinv + b_ref[...]
    h = jnp.maximum(agg, 0.0)
    o_ref[...] = (jnp.dot(h, w_ref[...], preferred_element_type=jnp.float32)
                  + bl_ref[...])


def _row_spec(width):
    return pl.BlockSpec((_BN, width), lambda i: (i, 0))


def _full_spec(r, cc):
    return pl.BlockSpec((r, cc), lambda i: (0, 0))


def _tc_first(x, w0, d0):
    return pl.pallas_call(
        _first_body,
        grid=(_GRID,),
        in_specs=[_row_spec(D_IN), _full_spec(D_IN, H), _row_spec(H)],
        out_specs=_row_spec(H),
        out_shape=jax.ShapeDtypeStruct((N_PAD, H), jnp.float32),
    )(x, w0, d0)


def _tc_mid(p0, tp, d0, b, w):
    return pl.pallas_call(
        _mid_body,
        grid=(_GRID,),
        in_specs=[_row_spec(H), _row_spec(H), _row_spec(H),
                  _full_spec(1, H), _full_spec(H, H)],
        out_specs=_row_spec(H),
        out_shape=jax.ShapeDtypeStruct((N_PAD, H), jnp.float32),
    )(p0, tp, d0, b, w)


def _tc_last(p0, tp, d0, b, wl, bl):
    return pl.pallas_call(
        _last_body,
        grid=(_GRID,),
        in_specs=[_row_spec(H), _row_spec(H), _row_spec(H),
                  _full_spec(1, H), _full_spec(H, NUM_OUT), _full_spec(1, NUM_OUT)],
        out_specs=_row_spec(NUM_OUT),
        out_shape=jax.ShapeDtypeStruct((N_PAD, NUM_OUT), jnp.float32),
    )(p0, tp, d0, b, wl, bl)


def _pad2(a, r, cc):
    return jnp.pad(a, ((0, r - a.shape[0]), (0, cc - a.shape[1])))


def kernel(x, edge_index, W0, b0, W_rest, b_rest, W_logits, b_logits):
    src = edge_index[0].astype(jnp.int32)
    dst = edge_index[1].astype(jnp.int32)
    npad = E_PAD - E
    pad_iota = lax.iota(jnp.int32, npad)
    src_p = jnp.concatenate([src, pad_iota % N]).reshape(E_PAD // BATCH, BATCH)
    dst_p = jnp.concatenate([dst, N + pad_iota % (N_PAD - N)]).reshape(
        E_PAD // BATCH, BATCH)

    z = jnp.zeros((AGG_R, H), jnp.float32)
    ones_t = jnp.ones((N_PAD, H), jnp.float32)

    d0 = _mp_full(ones_t, src_p, dst_p, z)

    xp = jnp.pad(x, ((0, N_PAD - N), (0, 0)))
    t = _tc_first(xp, W0, d0)

    biases = [b0] + [b_rest[i] for i in range(L - 1)]
    weights = [W_rest[i] for i in range(L - 1)]

    for i in range(L - 1):
        p = _mp_full(t, src_p, dst_p, z)
        t = _tc_mid(p, t, d0, biases[i].reshape(1, H), weights[i])

    p = _mp_full(t, src_p, dst_p, z)
    logits = _tc_last(p, t, d0, biases[L - 1].reshape(1, H),
                      W_logits, b_logits.reshape(1, NUM_OUT))
    return logits[:N]


# speedup vs baseline: 22.8382x; 1.2276x over previous
import functools

import jax
import jax.numpy as jnp
from jax import lax
from jax.experimental import pallas as pl
from jax.experimental.pallas import tpu as pltpu
from jax.experimental.pallas import tpu_sc as plsc

N = 10000
E = 320000
D_IN = 128
H = 64
HP = 128
NUM_OUT = 64
L = 10

NC = 2
NS = 16
N_PAD = 10240
RPT = N_PAD // NS

BATCH = 128
BT = 160
E_PAD = NS * BT * BATCH
CB = 4
CBB = CB * BATCH

_mesh = plsc.VectorSubcoreMesh(core_axis_name="c", subcore_axis_name="s", num_cores=NC)


HALFR = N_PAD // 2
NDUM = 256
AGG_R = HALFR + NDUM
RPT_A = AGG_R // NS


def _make_mp():
    @functools.partial(
        pl.kernel,
        mesh=_mesh,
        compiler_params=pltpu.CompilerParams(use_tc_tiling_on_sc=False, needs_layout_passes=False),
        out_type=(jax.ShapeDtypeStruct((HALFR, H), jnp.float32),
                  jax.ShapeDtypeStruct((HALFR, H), jnp.float32)),
        scratch_types=[
            pltpu.VMEM((BT + 3, BATCH), jnp.int32),
            pltpu.VMEM((BT + 3, BATCH), jnp.int32),
            pltpu.VMEM((2 * CB * BATCH, H), jnp.float32),
            pltpu.VMEM_SHARED((AGG_R, H), jnp.float32),
            pltpu.SemaphoreType.DMA,
            pltpu.SemaphoreType.DMA,
        ],
    )
    def _mp_kernel(t_hbm, src_hbm, dst_hbm, z_hbm, out0_hbm, out1_hbm,
                   src_v, dst_v, rows_v, agg_sh, sem_g, sem_s):
        c = lax.axis_index("c")
        s = lax.axis_index("s")
        pltpu.sync_copy(z_hbm.at[pl.ds(s * RPT_A, RPT_A)],
                        agg_sh.at[pl.ds(s * RPT_A, RPT_A)])
        pltpu.sync_copy(src_hbm.at[pl.ds(s * BT, BT)], src_v.at[pl.ds(0, BT)])
        pltpu.sync_copy(dst_hbm.at[pl.ds(s * BT, BT)], dst_v.at[pl.ds(0, BT)])

        base = c * HALFR
        iota16 = lax.iota(jnp.int32, 16)

        def compact(b, cnt):
            for j in range(BATCH // 16):
                v_dst = dst_v[b, pl.ds(j * 16, 16)]
                v_src = src_v[b, pl.ds(j * 16, 16)]
                local = v_dst - base
                m = (local >= 0) & (local < HALFR)
                mi = jnp.where(m, jnp.int32(1), jnp.int32(0))
                pos = cnt + plsc.cumsum(mi) - 1
                r = lax.shift_right_logical(pos, 7)
                q = pos & (BATCH - 1)
                plsc.store_scatter(dst_v, [r, q], local, mask=m)
                plsc.store_scatter(src_v, [r, q], v_src, mask=m)
                cnt = cnt + jnp.sum(mi)
            return cnt

        cnt = lax.fori_loop(0, BT, compact, jnp.int32(0))

        for j in range(CBB // 16):
            dummy = HALFR + ((iota16 + j * 16) & (NDUM - 1))
            tpos = cnt + j * 16 + iota16
            tr = lax.shift_right_logical(tpos, 7)
            tq = tpos & (BATCH - 1)
            plsc.store_scatter(dst_v, [tr, tq], dummy)
            plsc.store_scatter(src_v, [tr, tq], (iota16 + j * 16) * 8)

        nb = (cnt + CBB - 1) // CBB
        plsc.subcore_barrier()

        def _slot(buf, j):
            return rows_v.at[pl.ds((buf * CB + j) * BATCH, BATCH)]

        def fire_g(buf, rnd):
            for j in range(CB):
                pltpu.async_copy(t_hbm.at[src_v.at[rnd * CB + j]],
                                 _slot(buf, j), sem_g)

        def wait_g(buf, rnd):
            for j in range(CB):
                pltpu.make_async_copy(t_hbm.at[src_v.at[rnd * CB + j]],
                                      _slot(buf, j), sem_g).wait()

        def fire_s(buf, rnd):
            for j in range(CB):
                pltpu.async_copy(_slot(buf, j),
                                 agg_sh.at[dst_v.at[rnd * CB + j]], sem_s,
                                 add=True)

        def wait_s(buf, rnd):
            for j in range(CB):
                pltpu.make_async_copy(_slot(buf, j),
                                      agg_sh.at[dst_v.at[rnd * CB + j]],
                                      sem_s).wait()

        @pl.when(nb > 0)
        def _():
            fire_g(0, 0)

        def chunk(i, carry):
            cur = i & 1
            wait_g(cur, i)
            fire_s(cur, i)

            @pl.when(i > 0)
            def _():
                wait_s(1 - cur, i - 1)

            @pl.when(i + 1 < nb)
            def _():
                fire_g(1 - cur, i + 1)
            return carry

        lax.fori_loop(0, nb, chunk, 0)

        @pl.when(nb > 0)
        def _():
            wait_s((nb - 1) & 1, nb - 1)
        plsc.subcore_barrier()

        rps = HALFR // NS
        @pl.when(c == 0)
        def _():
            pltpu.sync_copy(agg_sh.at[pl.ds(s * rps, rps)],
                            out0_hbm.at[pl.ds(s * rps, rps)])

        @pl.when(c == 1)
        def _():
            pltpu.sync_copy(agg_sh.at[pl.ds(s * rps, rps)],
                            out1_hbm.at[pl.ds(s * rps, rps)])

    return _mp_kernel


_mp = _make_mp()


def _mp_full(t, src_p, dst_p, z):
    h0, h1 = _mp(t, src_p, dst_p, z)
    return jnp.concatenate([h0, h1], axis=0)


_BN = 2048
_GRID = N_PAD // _BN


def _dinv_block(d0):
    deg = d0[:, 0:1] + 1.0
    return lax.rsqrt(deg)


_ZBLK = (_BN, H)


def _first_body(x_ref, w_ref, d0_ref, o_ref):
    dinv = _dinv_block(d0_ref[...])
    h = jnp.dot(x_ref[...], w_ref[...], preferred_element_type=jnp.float32)
    o_ref[...] = h * dinv


def _mid_body(p0_ref, tp_ref, d0_ref, b_ref, w_ref, o_ref):
    dinv = _dinv_block(d0_ref[...])
    agg = (p0_ref[...] + tp_ref[...]) * dinv + b_ref[...]
    h = jnp.maximum(agg, 0.0)
    o_ref[...] = jnp.dot(h, w_ref[...], preferred_element_type=jnp.float32) * dinv


def _last_body(p0_ref, tp_ref, d0_ref, b_ref, w_ref, bl_ref,
               o_ref):
    dinv = _dinv_block(d0_ref[...])
    agg = (p0_ref[...] + tp_ref[...]) * dinv + b_ref[...]
    h = jnp.maximum(agg, 0.0)
    o_ref[...] = (jnp.dot(h, w_ref[...], preferred_element_type=jnp.float32)
                  + bl_ref[...])


def _row_spec(width):
    return pl.BlockSpec((_BN, width), lambda i: (i, 0))


def _full_spec(r, cc):
    return pl.BlockSpec((r, cc), lambda i: (0, 0))


def _tc_first(x, w0, d0):
    return pl.pallas_call(
        _first_body,
        grid=(_GRID,),
        in_specs=[_row_spec(D_IN), _full_spec(D_IN, H), _row_spec(H)],
        out_specs=_row_spec(H),
        out_shape=jax.ShapeDtypeStruct((N_PAD, H), jnp.float32),
    )(x, w0, d0)


def _tc_mid(p0, tp, d0, b, w):
    return pl.pallas_call(
        _mid_body,
        grid=(_GRID,),
        in_specs=[_row_spec(H), _row_spec(H), _row_spec(H),
                  _full_spec(1, H), _full_spec(H, H)],
        out_specs=_row_spec(H),
        out_shape=jax.ShapeDtypeStruct((N_PAD, H), jnp.float32),
    )(p0, tp, d0, b, w)


def _tc_last(p0, tp, d0, b, wl, bl):
    return pl.pallas_call(
        _last_body,
        grid=(_GRID,),
        in_specs=[_row_spec(H), _row_spec(H), _row_spec(H),
                  _full_spec(1, H), _full_spec(H, NUM_OUT), _full_spec(1, NUM_OUT)],
        out_specs=_row_spec(NUM_OUT),
        out_shape=jax.ShapeDtypeStruct((N_PAD, NUM_OUT), jnp.float32),
    )(p0, tp, d0, b, wl, bl)


def _pad2(a, r, cc):
    return jnp.pad(a, ((0, r - a.shape[0]), (0, cc - a.shape[1])))


def kernel(x, edge_index, W0, b0, W_rest, b_rest, W_logits, b_logits):
    src = edge_index[0].astype(jnp.int32)
    dst = edge_index[1].astype(jnp.int32)
    npad = E_PAD - E
    pad_iota = lax.iota(jnp.int32, npad)
    src_p = jnp.concatenate([src, pad_iota % N]).reshape(E_PAD // BATCH, BATCH)
    dst_p = jnp.concatenate([dst, N + pad_iota % (N_PAD - N)]).reshape(
        E_PAD // BATCH, BATCH)

    z = jnp.zeros((AGG_R, H), jnp.float32)
    ones_t = jnp.ones((N_PAD, H), jnp.float32)

    d0 = _mp_full(ones_t, src_p, dst_p, z)

    xp = jnp.pad(x, ((0, N_PAD - N), (0, 0)))
    t = _tc_first(xp, W0, d0)

    biases = [b0] + [b_rest[i] for i in range(L - 1)]
    weights = [W_rest[i] for i in range(L - 1)]

    for i in range(L - 1):
        p = _mp_full(t, src_p, dst_p, z)
        t = _tc_mid(p, t, d0, biases[i].reshape(1, H), weights[i])

    p = _mp_full(t, src_p, dst_p, z)
    logits = _tc_last(p, t, d0, biases[L - 1].reshape(1, H),
                      W_logits, b_logits.reshape(1, NUM_OUT))
    return logits[:N]
